# Initial kernel scaffold; baseline (speedup 1.0000x reference)
import jax
import jax.numpy as jnp
from jax.experimental import pallas as pl


def kernel(x, edge_index, W_mu_0, b_mu_0, bias_mu_0, W_mu_1, b_mu_1, bias_mu_1, W_mu_2, b_mu_2, bias_mu_2, W_ls_0, b_ls_0, bias_ls_0, W_ls_1, b_ls_1, bias_ls_1, W_ls_2, b_ls_2, bias_ls_2):
    raise NotImplementedError("write your pallas kernel here")


# speedup vs baseline: 14.5661x; 14.5661x over previous
import functools

import jax
import jax.numpy as jnp
from jax import lax
from jax.experimental import pallas as pl
from jax.experimental.pallas import tpu as pltpu
from jax.experimental.pallas import tpu_sc as plsc

N = 10000
E = 320000
NC, NS = 2, 16
NW = NC * NS
K = 128
NCH = 80
EW = K * NCH
EPAD = NW * EW
NPAD = 10240
NROW = NPAD // NS

_MESH = plsc.VectorSubcoreMesh(
    core_axis_name="c", subcore_axis_name="s", num_cores=NC, num_subcores=NS)

_SC_PARAMS = pltpu.CompilerParams(use_tc_tiling_on_sc=False)

_HIGH = lax.Precision.HIGHEST


def _deg_body(dst_hbm, out_hbm, idx_v, ones_v, zeros_v, acc_sh, sem):
    cid = lax.axis_index("c")
    sid = lax.axis_index("s")
    wid = sid * NC + cid

    def _fill(i, _):
        zeros_v[pl.ds(i * 16, 16)] = jnp.zeros((16,), jnp.float32)
        return 0
    lax.fori_loop(0, NROW // 16, _fill, 0)
    for i in range(K // 16):
        ones_v[pl.ds(i * 16, 16)] = jnp.ones((16,), jnp.float32)

    pltpu.sync_copy(zeros_v, acc_sh.at[pl.ds(sid * NROW, NROW)])
    plsc.subcore_barrier()

    pltpu.sync_copy(dst_hbm.at[wid], idx_v)

    def _step(j, _):
        pltpu.sync_copy(ones_v, acc_sh.at[idx_v.at[j]], add=True)
        return 0
    lax.fori_loop(0, NCH, _step, 0)
    plsc.subcore_barrier()

    pltpu.sync_copy(acc_sh.at[pl.ds(sid * NROW, NROW)],
                    out_hbm.at[cid, pl.ds(sid * NROW, NROW)])


@functools.partial(
    pl.kernel,
    out_type=jax.ShapeDtypeStruct((NC, NPAD), jnp.float32),
    mesh=_MESH,
    compiler_params=_SC_PARAMS,
    scratch_types=[
        pltpu.VMEM((NCH, K), jnp.int32),
        pltpu.VMEM((K,), jnp.float32),
        pltpu.VMEM((NROW,), jnp.float32),
        pltpu.VMEM_SHARED((NPAD,), jnp.float32),
        pltpu.SemaphoreType.DMA,
    ],
)
def _deg_sc(dst_hbm, out_hbm, idx_v, ones_v, zeros_v, acc_sh, sem):
    _deg_body(dst_hbm, out_hbm, idx_v, ones_v, zeros_v, acc_sh, sem)


def _make_prop(C):
    def body(u_hbm, src_hbm, dst_hbm, out_hbm,
             sidx_v, didx_v, rows_v, acc_sh, gsem):
        cid = lax.axis_index("c")
        sid = lax.axis_index("s")
        wid = sid * NC + cid

        def _fill(i, _):
            for k in range(C // 16):
                rows_v[i, pl.ds(k * 16, 16)] = jnp.zeros((16,), jnp.float32)
            return 0
        lax.fori_loop(0, K, _fill, 0)
        for r in range(NROW // K):
            pltpu.sync_copy(rows_v, acc_sh.at[pl.ds(sid * NROW + r * K, K)])
        plsc.subcore_barrier()

        pltpu.sync_copy(src_hbm.at[wid], sidx_v)
        pltpu.sync_copy(dst_hbm.at[wid], didx_v)

        def _step(j, _):
            pltpu.async_copy(u_hbm.at[sidx_v.at[j]], rows_v, gsem).wait()
            pltpu.sync_copy(rows_v, acc_sh.at[didx_v.at[j]], add=True)
            return 0
        lax.fori_loop(0, NCH, _step, 0)
        plsc.subcore_barrier()

        for r in range(NROW // K):
            pltpu.sync_copy(acc_sh.at[pl.ds(sid * NROW + r * K, K)],
                            out_hbm.at[cid, pl.ds(sid * NROW + r * K, K)])

    return functools.partial(
        pl.kernel,
        out_type=jax.ShapeDtypeStruct((NC, NPAD, C), jnp.float32),
        mesh=_MESH,
        compiler_params=_SC_PARAMS,
        scratch_types=[
            pltpu.VMEM((NCH, K), jnp.int32),
            pltpu.VMEM((NCH, K), jnp.int32),
            pltpu.VMEM((K, C), jnp.float32),
            pltpu.VMEM_SHARED((NPAD, C), jnp.float32),
            pltpu.SemaphoreType.DMA,
        ],
    )(body)


_prop128 = _make_prop(128)
_prop64 = _make_prop(64)
_prop32 = _make_prop(32)


def _pre0_body(deg0_ref, deg1_ref, x_ref, wm_ref, wl_ref, u0_ref, dinv_ref):
    deg = deg0_ref[...] + deg1_ref[...] + 1.0
    dinv = lax.rsqrt(deg)
    x = x_ref[...]
    zm = jnp.dot(x, wm_ref[...], precision=_HIGH)
    zl = jnp.dot(x, wl_ref[...], precision=_HIGH)
    d64 = jnp.broadcast_to(dinv, (N, 64))
    u0_ref[:, 0:64] = zm * d64
    u0_ref[:, 64:128] = zl * d64
    dinv_ref[...] = jnp.broadcast_to(dinv, (N, 128))


_pre0_tc = pl.pallas_call(
    _pre0_body,
    out_shape=[jax.ShapeDtypeStruct((N, 128), jnp.float32),
               jax.ShapeDtypeStruct((N, 128), jnp.float32)],
)


def _make_mid(C, F2):
    F = C // 2

    def body(p_ref, u_ref, dinv_ref, bm_ref, sm_ref, bl_ref, sl_ref,
             wm_ref, wl_ref, out_ref):
        pp = p_ref[...]
        d = dinv_ref[:, 0:C]
        h = d * (pp[0, 0:N] + pp[1, 0:N] + u_ref[...])
        hm = jnp.maximum(h[:, 0:F] + bm_ref[...] + sm_ref[...], 0.0)
        hl = jnp.maximum(h[:, F:C] + bl_ref[...] + sl_ref[...], 0.0)
        zm = jnp.dot(hm, wm_ref[...], precision=_HIGH)
        zl = jnp.dot(hl, wl_ref[...], precision=_HIGH)
        dn = dinv_ref[:, 0:F2]
        out_ref[:, 0:F2] = zm * dn
        out_ref[:, F2:2 * F2] = zl * dn

    return pl.pallas_call(
        body, out_shape=jax.ShapeDtypeStruct((N, 2 * F2), jnp.float32))


_mid1_tc = _make_mid(128, 32)
_mid2_tc = _make_mid(64, 16)


def _final_body(p_ref, u_ref, dinv_ref, bm_ref, sm_ref, bl_ref, sl_ref,
                mu_ref, ls_ref):
    pp = p_ref[...]
    d = dinv_ref[:, 0:32]
    h = d * (pp[0, 0:N] + pp[1, 0:N] + u_ref[...])
    hm = jnp.maximum(h[:, 0:16] + bm_ref[...] + sm_ref[...], 0.0)
    hl = jnp.maximum(h[:, 16:32] + bl_ref[...] + sl_ref[...], 0.0)

    def _softmax(a):
        m = jnp.max(a, axis=1, keepdims=True)
        e = jnp.exp(a - m)
        return e / jnp.sum(e, axis=1, keepdims=True)

    mu_ref[...] = _softmax(hm)
    ls_ref[...] = _softmax(hl)


_final_tc = pl.pallas_call(
    _final_body,
    out_shape=[jax.ShapeDtypeStruct((N, 16), jnp.float32),
               jax.ShapeDtypeStruct((N, 16), jnp.float32)],
)


def kernel(x, edge_index,
           W_mu_0, b_mu_0, bias_mu_0, W_mu_1, b_mu_1, bias_mu_1,
           W_mu_2, b_mu_2, bias_mu_2,
           W_ls_0, b_ls_0, bias_ls_0, W_ls_1, b_ls_1, bias_ls_1,
           W_ls_2, b_ls_2, bias_ls_2):
    npad_e = EPAD - E
    src = jnp.concatenate(
        [edge_index[0], jnp.zeros((npad_e,), jnp.int32)]).reshape(NW, NCH, K)
    dst = jnp.concatenate(
        [edge_index[1], jnp.full((npad_e,), N, jnp.int32)]).reshape(NW, NCH, K)

    degp = _deg_sc(dst)
    deg0 = degp[0, :N].reshape(N, 1)
    deg1 = degp[1, :N].reshape(N, 1)

    u0, dinvb = _pre0_tc(deg0, deg1, x, W_mu_0, W_ls_0)
    p0 = _prop128(u0, src, dst)

    u1 = _mid1_tc(p0, u0, dinvb,
                  b_mu_0.reshape(1, 64), bias_mu_0.reshape(1, 1),
                  b_ls_0.reshape(1, 64), bias_ls_0.reshape(1, 1),
                  W_mu_1, W_ls_1)
    p1 = _prop64(u1, src, dst)

    u2 = _mid2_tc(p1, u1, dinvb,
                  b_mu_1.reshape(1, 32), bias_mu_1.reshape(1, 1),
                  b_ls_1.reshape(1, 32), bias_ls_1.reshape(1, 1),
                  W_mu_2, W_ls_2)
    p2 = _prop32(u2, src, dst)

    mu, logstd = _final_tc(p2, u2, dinvb,
                           b_mu_2.reshape(1, 16), bias_mu_2.reshape(1, 1),
                           b_ls_2.reshape(1, 16), bias_ls_2.reshape(1, 1))
    return (mu, logstd)
